# baseline (device time: 54974 ns/iter reference)
import jax
import jax.numpy as jnp
from jax import lax
from jax.experimental import pallas as pl
from jax.experimental.pallas import tpu as pltpu

N_DEV = 16
BLK = 64
N = 1024

_GELU_C = 0.7978845608028654


def kernel(x, w_mat):
    m, k_per = x.shape
    _, n = w_mat.shape

    def body(x_ref, w_ref, out_ref, acc_ref, send_ref, recv_ref,
             send_sems, recv_sems):
        my = lax.axis_index("i")
        left = lax.rem(my + N_DEV - 1, N_DEV)
        right = lax.rem(my + 1, N_DEV)

        barrier = pltpu.get_barrier_semaphore()
        for nbr in (left, right):
            pl.semaphore_signal(
                barrier, inc=1,
                device_id=(nbr,), device_id_type=pl.DeviceIdType.MESH,
            )
        pl.semaphore_wait(barrier, 2)

        xb = x_ref[...].astype(jnp.bfloat16)
        wb = w_ref[...].astype(jnp.bfloat16)
        acc_ref[...] = jnp.dot(xb, wb, preferred_element_type=jnp.float32)

        for s in range(N_DEV - 1):
            b = lax.rem(my + N_DEV - s - 1, N_DEV)
            chunk = acc_ref[pl.ds(b * BLK, BLK), :]
            if s > 0:
                chunk = chunk + recv_ref[s - 1].astype(jnp.float32)
            send_ref[s] = chunk.astype(jnp.bfloat16)
            rdma = pltpu.make_async_remote_copy(
                src_ref=send_ref.at[s],
                dst_ref=recv_ref.at[s],
                send_sem=send_sems.at[s],
                recv_sem=recv_sems.at[s],
                device_id=(right,),
                device_id_type=pl.DeviceIdType.MESH,
            )
            rdma.start()
            rdma.wait()

        final = (acc_ref[pl.ds(my * BLK, BLK), :]
                 + recv_ref[N_DEV - 2].astype(jnp.float32))
        out_ref[...] = 0.5 * final * (
            1.0 + jnp.tanh(_GELU_C * (final + 0.044715 * final * final * final))
        )

    return pl.pallas_call(
        body,
        out_shape=jax.ShapeDtypeStruct((BLK, n), jnp.float32),
        in_specs=[
            pl.BlockSpec(memory_space=pltpu.VMEM),
            pl.BlockSpec(memory_space=pltpu.VMEM),
        ],
        out_specs=pl.BlockSpec(memory_space=pltpu.VMEM),
        scratch_shapes=[
            pltpu.VMEM((m, n), jnp.float32),
            pltpu.VMEM((N_DEV - 1, BLK, n), jnp.bfloat16),
            pltpu.VMEM((N_DEV - 1, BLK, n), jnp.bfloat16),
            pltpu.SemaphoreType.DMA((N_DEV - 1,)),
            pltpu.SemaphoreType.DMA((N_DEV - 1,)),
        ],
        compiler_params=pltpu.CompilerParams(collective_id=0),
    )(x, w_mat)


# device time: 30148 ns/iter; 1.8235x vs baseline; 1.8235x over previous
import jax
import jax.numpy as jnp
from jax import lax
from jax.experimental import pallas as pl
from jax.experimental.pallas import tpu as pltpu

N_DEV = 16
NZ = 4
BLK = 64
HALF = 512
N = 1024

_GELU_C = 0.7978845608028654
_MESH = pl.DeviceIdType.MESH


def _gelu(v):
    return 0.5 * v * (1.0 + jnp.tanh(_GELU_C * (v + 0.044715 * v * v * v)))


def kernel(x, w_mat):
    m, k_per = x.shape
    _, n = w_mat.shape

    def body(x_ref, w_ref, out_ref, acc_ref,
             p1s_a, p1r_a, p1s_b, p1r_b,
             p2s_a, p2r_a, p2s_b, p2r_b,
             zs_r, zr_r, zs_l, zr_l,
             p1_ssem, p1_rsem, p2_ssem, p2_rsem,
             z_ssem_r, z_rsem_r, z_ssem_l, z_rsem_l):
        p = lax.axis_index("i")
        z = p // NZ
        c = lax.rem(p, NZ)
        cx = jnp.bitwise_xor(c, 1)
        cy = 3 - c
        cd = 3 - cx
        px = NZ * z + cx
        py = NZ * z + cy
        pzr = NZ * lax.rem(z + 1, NZ) + c
        pzl = NZ * lax.rem(z + 3, NZ) + c

        barrier = pltpu.get_barrier_semaphore()
        for nbr in (px, py, pzr, pzl):
            pl.semaphore_signal(barrier, inc=1, device_id=(nbr,),
                                device_id_type=_MESH)
        pl.semaphore_wait(barrier, 4)

        xb = x_ref[...].astype(jnp.bfloat16)
        wb = w_ref[...].astype(jnp.bfloat16)
        acc_ref[...] = jnp.dot(xb, wb, preferred_element_type=jnp.float32)

        oa1 = jnp.minimum(cx, cd)
        oa2 = jnp.maximum(cx, cd)
        ka1 = jnp.minimum(c, cy)
        ka2 = jnp.maximum(c, cy)
        ob1 = jnp.minimum(cy, cd)
        ob2 = jnp.maximum(cy, cd)
        kb1 = jnp.minimum(c, cx)
        kb2 = jnp.maximum(c, cx)
        for zi in range(NZ):
            base = zi * NZ * BLK
            p1s_a[pl.ds(2 * zi * BLK, BLK), :] = (
                acc_ref[pl.ds(base + oa1 * BLK, BLK), 0:HALF]
                .astype(jnp.bfloat16))
            p1s_a[pl.ds((2 * zi + 1) * BLK, BLK), :] = (
                acc_ref[pl.ds(base + oa2 * BLK, BLK), 0:HALF]
                .astype(jnp.bfloat16))
            p1s_b[pl.ds(2 * zi * BLK, BLK), :] = (
                acc_ref[pl.ds(base + ob1 * BLK, BLK), HALF:N]
                .astype(jnp.bfloat16))
            p1s_b[pl.ds((2 * zi + 1) * BLK, BLK), :] = (
                acc_ref[pl.ds(base + ob2 * BLK, BLK), HALF:N]
                .astype(jnp.bfloat16))
        rdma_a = pltpu.make_async_remote_copy(
            src_ref=p1s_a, dst_ref=p1r_a,
            send_sem=p1_ssem.at[0], recv_sem=p1_rsem.at[0],
            device_id=(px,), device_id_type=_MESH)
        rdma_b = pltpu.make_async_remote_copy(
            src_ref=p1s_b, dst_ref=p1r_b,
            send_sem=p1_ssem.at[1], recv_sem=p1_rsem.at[1],
            device_id=(py,), device_id_type=_MESH)
        rdma_a.start()
        rdma_b.start()
        rdma_a.wait()
        rdma_b.wait()
        for zi in range(NZ):
            base = zi * NZ * BLK
            for j, kc in ((0, ka1), (1, ka2)):
                r = pl.ds(base + kc * BLK, BLK)
                acc_ref[r, 0:HALF] = (
                    acc_ref[r, 0:HALF]
                    + p1r_a[pl.ds((2 * zi + j) * BLK, BLK), :]
                    .astype(jnp.float32))
            for j, kc in ((0, kb1), (1, kb2)):
                r = pl.ds(base + kc * BLK, BLK)
                acc_ref[r, HALF:N] = (
                    acc_ref[r, HALF:N]
                    + p1r_b[pl.ds((2 * zi + j) * BLK, BLK), :]
                    .astype(jnp.float32))

        for zi in range(NZ):
            base = zi * NZ * BLK
            p2s_a[pl.ds(zi * BLK, BLK), :] = (
                acc_ref[pl.ds(base + cy * BLK, BLK), 0:HALF]
                .astype(jnp.bfloat16))
            p2s_b[pl.ds(zi * BLK, BLK), :] = (
                acc_ref[pl.ds(base + cx * BLK, BLK), HALF:N]
                .astype(jnp.bfloat16))
        rdma_a = pltpu.make_async_remote_copy(
            src_ref=p2s_a, dst_ref=p2r_a,
            send_sem=p2_ssem.at[0], recv_sem=p2_rsem.at[0],
            device_id=(py,), device_id_type=_MESH)
        rdma_b = pltpu.make_async_remote_copy(
            src_ref=p2s_b, dst_ref=p2r_b,
            send_sem=p2_ssem.at[1], recv_sem=p2_rsem.at[1],
            device_id=(px,), device_id_type=_MESH)
        rdma_a.start()
        rdma_b.start()
        rdma_a.wait()
        rdma_b.wait()
        for zi in range(NZ):
            r = pl.ds((zi * NZ + c) * BLK, BLK)
            acc_ref[r, 0:HALF] = (
                acc_ref[r, 0:HALF]
                + p2r_a[pl.ds(zi * BLK, BLK), :].astype(jnp.float32))
            acc_ref[r, HALF:N] = (
                acc_ref[r, HALF:N]
                + p2r_b[pl.ds(zi * BLK, BLK), :].astype(jnp.float32))

        for s in range(NZ - 1):
            br = lax.rem(z + NZ - s - 1, NZ)
            bl = lax.rem(z + s + 1, NZ)
            chunk_r = acc_ref[pl.ds((br * NZ + c) * BLK, BLK), 0:HALF]
            chunk_l = acc_ref[pl.ds((bl * NZ + c) * BLK, BLK), HALF:N]
            if s > 0:
                chunk_r = chunk_r + zr_r[s - 1].astype(jnp.float32)
                chunk_l = chunk_l + zr_l[s - 1].astype(jnp.float32)
            zs_r[s] = chunk_r.astype(jnp.bfloat16)
            zs_l[s] = chunk_l.astype(jnp.bfloat16)
            rdma_r = pltpu.make_async_remote_copy(
                src_ref=zs_r.at[s], dst_ref=zr_r.at[s],
                send_sem=z_ssem_r.at[s], recv_sem=z_rsem_r.at[s],
                device_id=(pzr,), device_id_type=_MESH)
            rdma_l = pltpu.make_async_remote_copy(
                src_ref=zs_l.at[s], dst_ref=zr_l.at[s],
                send_sem=z_ssem_l.at[s], recv_sem=z_rsem_l.at[s],
                device_id=(pzl,), device_id_type=_MESH)
            rdma_r.start()
            rdma_l.start()
            rdma_r.wait()
            rdma_l.wait()

        mine = pl.ds(p * BLK, BLK)
        fin_a = acc_ref[mine, 0:HALF] + zr_r[NZ - 2].astype(jnp.float32)
        fin_b = acc_ref[mine, HALF:N] + zr_l[NZ - 2].astype(jnp.float32)
        out_ref[:, 0:HALF] = _gelu(fin_a)
        out_ref[:, HALF:N] = _gelu(fin_b)

    return pl.pallas_call(
        body,
        out_shape=jax.ShapeDtypeStruct((BLK, n), jnp.float32),
        in_specs=[
            pl.BlockSpec(memory_space=pltpu.VMEM),
            pl.BlockSpec(memory_space=pltpu.VMEM),
        ],
        out_specs=pl.BlockSpec(memory_space=pltpu.VMEM),
        scratch_shapes=[
            pltpu.VMEM((m, n), jnp.float32),
            pltpu.VMEM((8 * BLK, HALF), jnp.bfloat16),
            pltpu.VMEM((8 * BLK, HALF), jnp.bfloat16),
            pltpu.VMEM((8 * BLK, HALF), jnp.bfloat16),
            pltpu.VMEM((8 * BLK, HALF), jnp.bfloat16),
            pltpu.VMEM((4 * BLK, HALF), jnp.bfloat16),
            pltpu.VMEM((4 * BLK, HALF), jnp.bfloat16),
            pltpu.VMEM((4 * BLK, HALF), jnp.bfloat16),
            pltpu.VMEM((4 * BLK, HALF), jnp.bfloat16),
            pltpu.VMEM((NZ - 1, BLK, HALF), jnp.bfloat16),
            pltpu.VMEM((NZ - 1, BLK, HALF), jnp.bfloat16),
            pltpu.VMEM((NZ - 1, BLK, HALF), jnp.bfloat16),
            pltpu.VMEM((NZ - 1, BLK, HALF), jnp.bfloat16),
            pltpu.SemaphoreType.DMA((2,)),
            pltpu.SemaphoreType.DMA((2,)),
            pltpu.SemaphoreType.DMA((2,)),
            pltpu.SemaphoreType.DMA((2,)),
            pltpu.SemaphoreType.DMA((NZ - 1,)),
            pltpu.SemaphoreType.DMA((NZ - 1,)),
            pltpu.SemaphoreType.DMA((NZ - 1,)),
            pltpu.SemaphoreType.DMA((NZ - 1,)),
        ],
        compiler_params=pltpu.CompilerParams(collective_id=0),
    )(x, w_mat)


# device time: 26099 ns/iter; 2.1064x vs baseline; 1.1551x over previous
import jax
import jax.numpy as jnp
from jax import lax
from jax.experimental import pallas as pl
from jax.experimental.pallas import tpu as pltpu

N_DEV = 16
NZ = 4
BLK = 64
HALF = 512
N = 1024

_GELU_C = 0.7978845608028654
_MESH = pl.DeviceIdType.MESH


def _gelu(v):
    return 0.5 * v * (1.0 + jnp.tanh(_GELU_C * (v + 0.044715 * v * v * v)))


def kernel(x, w_mat):
    m, k_per = x.shape
    _, n = w_mat.shape

    def body(x_ref, w_ref, out_ref, acc_ref,
             p1s_a, p1r_a, p1s_b, p1r_b,
             p2s_a, p2r_a, p2s_b, p2r_b,
             zds, zdr,
             p1_ssem, p1_rsem, p2_ssem, p2_rsem,
             z_ssem, z_rsem):
        p = lax.axis_index("i")
        z = p // NZ
        c = lax.rem(p, NZ)
        cx = jnp.bitwise_xor(c, 1)
        cy = 3 - c
        cd = 3 - cx
        px = NZ * z + cx
        py = NZ * z + cy
        z_dests = []
        for k in range(NZ - 1):
            zd = k + (k >= z).astype(jnp.int32)
            z_dests.append((NZ * zd + c, jnp.where(z > zd, z - 1, z)))

        barrier = pltpu.get_barrier_semaphore()
        for nbr in (px, py) + tuple(d for d, _ in z_dests):
            pl.semaphore_signal(barrier, inc=1, device_id=(nbr,),
                                device_id_type=_MESH)
        pl.semaphore_wait(barrier, 5)

        xb = x_ref[...].astype(jnp.bfloat16)
        wb = w_ref[...].astype(jnp.bfloat16)
        acc_ref[...] = jnp.dot(xb, wb, preferred_element_type=jnp.float32)

        oa1 = jnp.minimum(cx, cd)
        oa2 = jnp.maximum(cx, cd)
        ka1 = jnp.minimum(c, cy)
        ka2 = jnp.maximum(c, cy)
        ob1 = jnp.minimum(cy, cd)
        ob2 = jnp.maximum(cy, cd)
        kb1 = jnp.minimum(c, cx)
        kb2 = jnp.maximum(c, cx)
        for zi in range(NZ):
            base = zi * NZ * BLK
            p1s_a[pl.ds(2 * zi * BLK, BLK), :] = (
                acc_ref[pl.ds(base + oa1 * BLK, BLK), 0:HALF]
                .astype(jnp.bfloat16))
            p1s_a[pl.ds((2 * zi + 1) * BLK, BLK), :] = (
                acc_ref[pl.ds(base + oa2 * BLK, BLK), 0:HALF]
                .astype(jnp.bfloat16))
            p1s_b[pl.ds(2 * zi * BLK, BLK), :] = (
                acc_ref[pl.ds(base + ob1 * BLK, BLK), HALF:N]
                .astype(jnp.bfloat16))
            p1s_b[pl.ds((2 * zi + 1) * BLK, BLK), :] = (
                acc_ref[pl.ds(base + ob2 * BLK, BLK), HALF:N]
                .astype(jnp.bfloat16))
        rdma_a = pltpu.make_async_remote_copy(
            src_ref=p1s_a, dst_ref=p1r_a,
            send_sem=p1_ssem.at[0], recv_sem=p1_rsem.at[0],
            device_id=(px,), device_id_type=_MESH)
        rdma_b = pltpu.make_async_remote_copy(
            src_ref=p1s_b, dst_ref=p1r_b,
            send_sem=p1_ssem.at[1], recv_sem=p1_rsem.at[1],
            device_id=(py,), device_id_type=_MESH)
        rdma_a.start()
        rdma_b.start()
        rdma_a.wait()
        rdma_b.wait()
        for zi in range(NZ):
            base = zi * NZ * BLK
            for j, kc in ((0, ka1), (1, ka2)):
                r = pl.ds(base + kc * BLK, BLK)
                acc_ref[r, 0:HALF] = (
                    acc_ref[r, 0:HALF]
                    + p1r_a[pl.ds((2 * zi + j) * BLK, BLK), :]
                    .astype(jnp.float32))
            for j, kc in ((0, kb1), (1, kb2)):
                r = pl.ds(base + kc * BLK, BLK)
                acc_ref[r, HALF:N] = (
                    acc_ref[r, HALF:N]
                    + p1r_b[pl.ds((2 * zi + j) * BLK, BLK), :]
                    .astype(jnp.float32))

        for zi in range(NZ):
            base = zi * NZ * BLK
            p2s_a[pl.ds(zi * BLK, BLK), :] = (
                acc_ref[pl.ds(base + cy * BLK, BLK), 0:HALF]
                .astype(jnp.bfloat16))
            p2s_b[pl.ds(zi * BLK, BLK), :] = (
                acc_ref[pl.ds(base + cx * BLK, BLK), HALF:N]
                .astype(jnp.bfloat16))
        rdma_a = pltpu.make_async_remote_copy(
            src_ref=p2s_a, dst_ref=p2r_a,
            send_sem=p2_ssem.at[0], recv_sem=p2_rsem.at[0],
            device_id=(py,), device_id_type=_MESH)
        rdma_b = pltpu.make_async_remote_copy(
            src_ref=p2s_b, dst_ref=p2r_b,
            send_sem=p2_ssem.at[1], recv_sem=p2_rsem.at[1],
            device_id=(px,), device_id_type=_MESH)
        rdma_a.start()
        rdma_b.start()
        rdma_a.wait()
        rdma_b.wait()
        for zi in range(NZ):
            r = pl.ds((zi * NZ + c) * BLK, BLK)
            acc_ref[r, 0:HALF] = (
                acc_ref[r, 0:HALF]
                + p2r_a[pl.ds(zi * BLK, BLK), :].astype(jnp.float32))
            acc_ref[r, HALF:N] = (
                acc_ref[r, HALF:N]
                + p2r_b[pl.ds(zi * BLK, BLK), :].astype(jnp.float32))

        sends = []
        for k, (pzd, slot) in enumerate(z_dests):
            zd = pzd // NZ
            zds[k] = (acc_ref[pl.ds((zd * NZ + c) * BLK, BLK), :]
                      .astype(jnp.bfloat16))
            r = pltpu.make_async_remote_copy(
                src_ref=zds.at[k], dst_ref=zdr.at[slot],
                send_sem=z_ssem.at[k], recv_sem=z_rsem.at[slot],
                device_id=(pzd,), device_id_type=_MESH)
            r.start()
            sends.append(r)
        for j in range(NZ - 1):
            pltpu.make_async_remote_copy(
                src_ref=zds.at[j], dst_ref=zdr.at[j],
                send_sem=z_ssem.at[j], recv_sem=z_rsem.at[j],
                device_id=(p,), device_id_type=_MESH).wait_recv()
        for r in sends:
            r.wait_send()

        fin = (acc_ref[pl.ds(p * BLK, BLK), :]
               + zdr[0].astype(jnp.float32)
               + zdr[1].astype(jnp.float32)
               + zdr[2].astype(jnp.float32))
        out_ref[...] = _gelu(fin)

    return pl.pallas_call(
        body,
        out_shape=jax.ShapeDtypeStruct((BLK, n), jnp.float32),
        in_specs=[
            pl.BlockSpec(memory_space=pltpu.VMEM),
            pl.BlockSpec(memory_space=pltpu.VMEM),
        ],
        out_specs=pl.BlockSpec(memory_space=pltpu.VMEM),
        scratch_shapes=[
            pltpu.VMEM((m, n), jnp.float32),
            pltpu.VMEM((8 * BLK, HALF), jnp.bfloat16),
            pltpu.VMEM((8 * BLK, HALF), jnp.bfloat16),
            pltpu.VMEM((8 * BLK, HALF), jnp.bfloat16),
            pltpu.VMEM((8 * BLK, HALF), jnp.bfloat16),
            pltpu.VMEM((4 * BLK, HALF), jnp.bfloat16),
            pltpu.VMEM((4 * BLK, HALF), jnp.bfloat16),
            pltpu.VMEM((4 * BLK, HALF), jnp.bfloat16),
            pltpu.VMEM((4 * BLK, HALF), jnp.bfloat16),
            pltpu.VMEM((NZ - 1, BLK, N), jnp.bfloat16),
            pltpu.VMEM((NZ - 1, BLK, N), jnp.bfloat16),
            pltpu.SemaphoreType.DMA((2,)),
            pltpu.SemaphoreType.DMA((2,)),
            pltpu.SemaphoreType.DMA((2,)),
            pltpu.SemaphoreType.DMA((2,)),
            pltpu.SemaphoreType.DMA((NZ - 1,)),
            pltpu.SemaphoreType.DMA((NZ - 1,)),
        ],
        compiler_params=pltpu.CompilerParams(collective_id=0),
    )(x, w_mat)


# device time: 24644 ns/iter; 2.2307x vs baseline; 1.0590x over previous
import jax
import jax.numpy as jnp
from jax import lax
from jax.experimental import pallas as pl
from jax.experimental.pallas import tpu as pltpu

N_DEV = 16
NZ = 4
BLK = 64
HALF = 512
N = 1024

_GELU_C = 0.7978845608028654
_MESH = pl.DeviceIdType.MESH


def _gelu(v):
    return 0.5 * v * (1.0 + jnp.tanh(_GELU_C * (v + 0.044715 * v * v * v)))


def kernel(x, w_mat):
    m, k_per = x.shape
    _, n = w_mat.shape

    def body(x_ref, w_ref, out_ref, acc_ref,
             p1s_a, p1r_a, p1s_b, p1r_b,
             p2s_a, p2r_a, p2s_b, p2r_b,
             zfull, zdr,
             p1_ssem, p1_rsem, p2_ssem, p2_rsem,
             z_ssem, z_rsem):
        p = lax.axis_index("i")
        z = p // NZ
        c = lax.rem(p, NZ)
        cx = jnp.bitwise_xor(c, 1)
        cy = 3 - c
        cd = 3 - cx
        px = NZ * z + cx
        py = NZ * z + cy

        z_dests = []
        for k in range(NZ - 1):
            zd = k + (k >= z).astype(jnp.int32)
            z_dests.append((zd, NZ * zd + c, jnp.where(z > zd, z - 1, z)))

        barrier = pltpu.get_barrier_semaphore()
        for nbr in (px, py) + tuple(d for _, d, _ in z_dests):
            pl.semaphore_signal(barrier, inc=1, device_id=(nbr,),
                                device_id_type=_MESH)
        pl.semaphore_wait(barrier, 5)

        xb = x_ref[...].astype(jnp.bfloat16)
        wb = w_ref[...].astype(jnp.bfloat16)

        oa1 = jnp.minimum(cx, cd)
        oa2 = jnp.maximum(cx, cd)
        ob1 = jnp.minimum(cy, cd)
        ob2 = jnp.maximum(cy, cd)
        ja_c = (c > cy).astype(jnp.int32)
        ja_f = 1 - ja_c
        jb_c = (c > cx).astype(jnp.int32)
        jb_f = 1 - jb_c

        acc_ref[:, 0:HALF] = jnp.dot(xb, wb[:, 0:HALF],
                                     preferred_element_type=jnp.float32)
        for zi in range(NZ):
            base = zi * NZ * BLK
            p1s_a[pl.ds(2 * zi * BLK, BLK), :] = (
                acc_ref[pl.ds(base + oa1 * BLK, BLK), 0:HALF]
                .astype(jnp.bfloat16))
            p1s_a[pl.ds((2 * zi + 1) * BLK, BLK), :] = (
                acc_ref[pl.ds(base + oa2 * BLK, BLK), 0:HALF]
                .astype(jnp.bfloat16))
        rdma_a1 = pltpu.make_async_remote_copy(
            src_ref=p1s_a, dst_ref=p1r_a,
            send_sem=p1_ssem.at[0], recv_sem=p1_rsem.at[0],
            device_id=(px,), device_id_type=_MESH)
        rdma_a1.start()

        acc_ref[:, HALF:N] = jnp.dot(xb, wb[:, HALF:N],
                                     preferred_element_type=jnp.float32)
        for zi in range(NZ):
            base = zi * NZ * BLK
            p1s_b[pl.ds(2 * zi * BLK, BLK), :] = (
                acc_ref[pl.ds(base + ob1 * BLK, BLK), HALF:N]
                .astype(jnp.bfloat16))
            p1s_b[pl.ds((2 * zi + 1) * BLK, BLK), :] = (
                acc_ref[pl.ds(base + ob2 * BLK, BLK), HALF:N]
                .astype(jnp.bfloat16))
        rdma_b1 = pltpu.make_async_remote_copy(
            src_ref=p1s_b, dst_ref=p1r_b,
            send_sem=p1_ssem.at[1], recv_sem=p1_rsem.at[1],
            device_id=(py,), device_id_type=_MESH)
        rdma_b1.start()

        rdma_a1.wait()
        for zi in range(NZ):
            base = zi * NZ * BLK
            rc = pl.ds(base + c * BLK, BLK)
            acc_ref[rc, 0:HALF] = (
                acc_ref[rc, 0:HALF]
                + p1r_a[pl.ds((2 * zi + ja_c) * BLK, BLK), :]
                .astype(jnp.float32))
            p2s_a[pl.ds(zi * BLK, BLK), :] = (
                acc_ref[pl.ds(base + cy * BLK, BLK), 0:HALF]
                + p1r_a[pl.ds((2 * zi + ja_f) * BLK, BLK), :]
                .astype(jnp.float32)).astype(jnp.bfloat16)
        rdma_a2 = pltpu.make_async_remote_copy(
            src_ref=p2s_a, dst_ref=p2r_a,
            send_sem=p2_ssem.at[0], recv_sem=p2_rsem.at[0],
            device_id=(py,), device_id_type=_MESH)
        rdma_a2.start()

        rdma_b1.wait()
        for zi in range(NZ):
            base = zi * NZ * BLK
            rc = pl.ds(base + c * BLK, BLK)
            acc_ref[rc, HALF:N] = (
                acc_ref[rc, HALF:N]
                + p1r_b[pl.ds((2 * zi + jb_c) * BLK, BLK), :]
                .astype(jnp.float32))
            p2s_b[pl.ds(zi * BLK, BLK), :] = (
                acc_ref[pl.ds(base + cx * BLK, BLK), HALF:N]
                + p1r_b[pl.ds((2 * zi + jb_f) * BLK, BLK), :]
                .astype(jnp.float32)).astype(jnp.bfloat16)
        rdma_b2 = pltpu.make_async_remote_copy(
            src_ref=p2s_b, dst_ref=p2r_b,
            send_sem=p2_ssem.at[1], recv_sem=p2_rsem.at[1],
            device_id=(px,), device_id_type=_MESH)
        rdma_b2.start()

        rdma_a2.wait()
        for zi in range(NZ):
            rc = pl.ds((zi * NZ + c) * BLK, BLK)
            zfull[zi, :, 0:HALF] = (
                acc_ref[rc, 0:HALF]
                + p2r_a[pl.ds(zi * BLK, BLK), :].astype(jnp.float32)
            ).astype(jnp.bfloat16)
        rdma_b2.wait()
        for zi in range(NZ):
            rc = pl.ds((zi * NZ + c) * BLK, BLK)
            zfull[zi, :, HALF:N] = (
                acc_ref[rc, HALF:N]
                + p2r_b[pl.ds(zi * BLK, BLK), :].astype(jnp.float32)
            ).astype(jnp.bfloat16)

        sends = []
        for k, (zd, pzd, slot) in enumerate(z_dests):
            r = pltpu.make_async_remote_copy(
                src_ref=zfull.at[zd], dst_ref=zdr.at[slot],
                send_sem=z_ssem.at[k], recv_sem=z_rsem.at[slot],
                device_id=(pzd,), device_id_type=_MESH)
            r.start()
            sends.append(r)

        fin = zfull[z].astype(jnp.float32)
        for j in range(NZ - 1):
            pltpu.make_async_remote_copy(
                src_ref=zfull.at[j], dst_ref=zdr.at[j],
                send_sem=z_ssem.at[j], recv_sem=z_rsem.at[j],
                device_id=(p,), device_id_type=_MESH).wait_recv()
            fin = fin + zdr[j].astype(jnp.float32)
        out_ref[...] = _gelu(fin)
        for r in sends:
            r.wait_send()

    return pl.pallas_call(
        body,
        out_shape=jax.ShapeDtypeStruct((BLK, n), jnp.float32),
        in_specs=[
            pl.BlockSpec(memory_space=pltpu.VMEM),
            pl.BlockSpec(memory_space=pltpu.VMEM),
        ],
        out_specs=pl.BlockSpec(memory_space=pltpu.VMEM),
        scratch_shapes=[
            pltpu.VMEM((m, n), jnp.float32),
            pltpu.VMEM((8 * BLK, HALF), jnp.bfloat16),
            pltpu.VMEM((8 * BLK, HALF), jnp.bfloat16),
            pltpu.VMEM((8 * BLK, HALF), jnp.bfloat16),
            pltpu.VMEM((8 * BLK, HALF), jnp.bfloat16),
            pltpu.VMEM((4 * BLK, HALF), jnp.bfloat16),
            pltpu.VMEM((4 * BLK, HALF), jnp.bfloat16),
            pltpu.VMEM((4 * BLK, HALF), jnp.bfloat16),
            pltpu.VMEM((4 * BLK, HALF), jnp.bfloat16),
            pltpu.VMEM((NZ, BLK, N), jnp.bfloat16),
            pltpu.VMEM((NZ - 1, BLK, N), jnp.bfloat16),
            pltpu.SemaphoreType.DMA((2,)),
            pltpu.SemaphoreType.DMA((2,)),
            pltpu.SemaphoreType.DMA((2,)),
            pltpu.SemaphoreType.DMA((2,)),
            pltpu.SemaphoreType.DMA((NZ - 1,)),
            pltpu.SemaphoreType.DMA((NZ - 1,)),
        ],
        compiler_params=pltpu.CompilerParams(collective_id=0),
    )(x, w_mat)


# device time: 21995 ns/iter; 2.4994x vs baseline; 1.1204x over previous
import jax
import jax.numpy as jnp
from jax import lax
from jax.experimental import pallas as pl
from jax.experimental.pallas import tpu as pltpu

N_DEV = 16
NZ = 4
BLK = 64
N = 1024
NS = 4
SW = N // NS

_GELU_C = 0.7978845608028654
_MESH = pl.DeviceIdType.MESH


def _gelu(v):
    return 0.5 * v * (1.0 + jnp.tanh(_GELU_C * (v + 0.044715 * v * v * v)))


def kernel(x, w_mat):
    m, k_per = x.shape
    _, n = w_mat.shape

    def body(x_ref, w_ref, out_ref, acc_ref,
             p1s, p1r, p2s, p2r, zfull, zdr,
             p1_ssem, p1_rsem, p2_ssem, p2_rsem,
             z_ssem, z_rsem):
        p = lax.axis_index("i")
        z = p // NZ
        c = lax.rem(p, NZ)
        cx = jnp.bitwise_xor(c, 1)
        cy = 3 - c
        cd = 3 - cx
        px = NZ * z + cx
        py = NZ * z + cy

        z_dests = []
        for k in range(NZ - 1):
            zd = k + (k >= z).astype(jnp.int32)
            z_dests.append((zd, NZ * zd + c, jnp.where(z > zd, z - 1, z)))

        barrier = pltpu.get_barrier_semaphore()
        for nbr in (px, py) + tuple(d for _, d, _ in z_dests):
            pl.semaphore_signal(barrier, inc=1, device_id=(nbr,),
                                device_id_type=_MESH)
        pl.semaphore_wait(barrier, 5)

        xb = x_ref[...].astype(jnp.bfloat16)
        wb = w_ref[...].astype(jnp.bfloat16)

        def strip_cfg(s):
            if s % 2 == 0:
                o1, o2 = jnp.minimum(cx, cd), jnp.maximum(cx, cd)
                fwd = cy
                return (px, py), (o1, o2), fwd
            o1, o2 = jnp.minimum(cy, cd), jnp.maximum(cy, cd)
            return (py, px), (o1, o2), cx

        p1_rdmas, p2_rdmas, z_sends = [], [], []

        for s in range(NS):
            cols = slice(s * SW, (s + 1) * SW)
            (peer1, _), (o1, o2), _ = strip_cfg(s)
            acc_ref[:, cols] = jnp.dot(xb, wb[:, cols],
                                       preferred_element_type=jnp.float32)
            for zi in range(NZ):
                base = zi * NZ * BLK
                p1s[s, pl.ds(2 * zi * BLK, BLK), :] = (
                    acc_ref[pl.ds(base + o1 * BLK, BLK), cols]
                    .astype(jnp.bfloat16))
                p1s[s, pl.ds((2 * zi + 1) * BLK, BLK), :] = (
                    acc_ref[pl.ds(base + o2 * BLK, BLK), cols]
                    .astype(jnp.bfloat16))
            r = pltpu.make_async_remote_copy(
                src_ref=p1s.at[s], dst_ref=p1r.at[s],
                send_sem=p1_ssem.at[s], recv_sem=p1_rsem.at[s],
                device_id=(peer1,), device_id_type=_MESH)
            r.start()
            p1_rdmas.append(r)

        for s in range(NS):
            cols = slice(s * SW, (s + 1) * SW)
            (_, peer2), _, fwd = strip_cfg(s)
            j_c = (c > fwd).astype(jnp.int32)
            j_f = 1 - j_c
            p1_rdmas[s].wait()
            for zi in range(NZ):
                base = zi * NZ * BLK
                rc = pl.ds(base + c * BLK, BLK)
                acc_ref[rc, cols] = (
                    acc_ref[rc, cols]
                    + p1r[s, pl.ds((2 * zi + j_c) * BLK, BLK), :]
                    .astype(jnp.float32))
                p2s[s, pl.ds(zi * BLK, BLK), :] = (
                    acc_ref[pl.ds(base + fwd * BLK, BLK), cols]
                    + p1r[s, pl.ds((2 * zi + j_f) * BLK, BLK), :]
                    .astype(jnp.float32)).astype(jnp.bfloat16)
            r = pltpu.make_async_remote_copy(
                src_ref=p2s.at[s], dst_ref=p2r.at[s],
                send_sem=p2_ssem.at[s], recv_sem=p2_rsem.at[s],
                device_id=(peer2,), device_id_type=_MESH)
            r.start()
            p2_rdmas.append(r)

        for s in range(NS):
            cols = slice(s * SW, (s + 1) * SW)
            p2_rdmas[s].wait()
            for zi in range(NZ):
                rc = pl.ds((zi * NZ + c) * BLK, BLK)
                zfull[zi, :, cols] = (
                    acc_ref[rc, cols]
                    + p2r[s, pl.ds(zi * BLK, BLK), :].astype(jnp.float32)
                ).astype(jnp.bfloat16)
            for k, (zd, pzd, slot) in enumerate(z_dests):
                r = pltpu.make_async_remote_copy(
                    src_ref=zfull.at[zd, :, pl.ds(s * SW, SW)],
                    dst_ref=zdr.at[slot, :, pl.ds(s * SW, SW)],
                    send_sem=z_ssem.at[k * NS + s],
                    recv_sem=z_rsem.at[slot * NS + s],
                    device_id=(pzd,), device_id_type=_MESH)
                r.start()
                z_sends.append(r)

        for s in range(NS):
            cols = slice(s * SW, (s + 1) * SW)
            fin = zfull[z, :, cols].astype(jnp.float32)
            for j in range(NZ - 1):
                pltpu.make_async_remote_copy(
                    src_ref=zfull.at[j, :, pl.ds(s * SW, SW)],
                    dst_ref=zdr.at[j, :, pl.ds(s * SW, SW)],
                    send_sem=z_ssem.at[j * NS + s],
                    recv_sem=z_rsem.at[j * NS + s],
                    device_id=(p,), device_id_type=_MESH).wait_recv()
                fin = fin + zdr[j, :, cols].astype(jnp.float32)
            out_ref[:, cols] = _gelu(fin)
        for r in z_sends:
            r.wait_send()

    return pl.pallas_call(
        body,
        out_shape=jax.ShapeDtypeStruct((BLK, n), jnp.float32),
        in_specs=[
            pl.BlockSpec(memory_space=pltpu.VMEM),
            pl.BlockSpec(memory_space=pltpu.VMEM),
        ],
        out_specs=pl.BlockSpec(memory_space=pltpu.VMEM),
        scratch_shapes=[
            pltpu.VMEM((m, n), jnp.float32),
            pltpu.VMEM((NS, 8 * BLK, SW), jnp.bfloat16),
            pltpu.VMEM((NS, 8 * BLK, SW), jnp.bfloat16),
            pltpu.VMEM((NS, 4 * BLK, SW), jnp.bfloat16),
            pltpu.VMEM((NS, 4 * BLK, SW), jnp.bfloat16),
            pltpu.VMEM((NZ, BLK, N), jnp.bfloat16),
            pltpu.VMEM((NZ - 1, BLK, N), jnp.bfloat16),
            pltpu.SemaphoreType.DMA((NS,)),
            pltpu.SemaphoreType.DMA((NS,)),
            pltpu.SemaphoreType.DMA((NS,)),
            pltpu.SemaphoreType.DMA((NS,)),
            pltpu.SemaphoreType.DMA(((NZ - 1) * NS,)),
            pltpu.SemaphoreType.DMA(((NZ - 1) * NS,)),
        ],
        compiler_params=pltpu.CompilerParams(collective_id=0),
    )(x, w_mat)


# device time: 21404 ns/iter; 2.5684x vs baseline; 1.0276x over previous
import jax
import jax.numpy as jnp
from jax import lax
from jax.experimental import pallas as pl
from jax.experimental.pallas import tpu as pltpu

N_DEV = 16
NZ = 4
BLK = 64
N = 1024
NS = 3
SW = 256
CCOL = NS * SW
PG = NZ * BLK

_GELU_C = 0.7978845608028654
_MESH = pl.DeviceIdType.MESH


def _gelu(v):
    return 0.5 * v * (1.0 + jnp.tanh(_GELU_C * (v + 0.044715 * v * v * v)))


def kernel(x, w_mat):
    m, k_per = x.shape
    _, n = w_mat.shape

    def body(x_ref, w_ref, out_ref, acc_ref,
             p1s, p1r, p2s, p2r, zfull, zdr,
             czs, czr, cp1s, cp1r, cp2s, cp2r,
             p1_ssem, p1_rsem, p2_ssem, p2_rsem, z_ssem, z_rsem,
             cz_ssem, cz_rsem, cp1_ssem, cp1_rsem, cp2_ssem, cp2_rsem):
        p = lax.axis_index("i")
        z = p // NZ
        c = lax.rem(p, NZ)
        cx = jnp.bitwise_xor(c, 1)
        cy = 3 - c
        cd = 3 - cx
        px = NZ * z + cx
        py = NZ * z + cy

        z_dests = []
        for k in range(NZ - 1):
            zd = k + (k >= z).astype(jnp.int32)
            z_dests.append((zd, NZ * zd + c, jnp.where(z > zd, z - 1, z)))

        barrier = pltpu.get_barrier_semaphore()
        for nbr in (px, py) + tuple(d for _, d, _ in z_dests):
            pl.semaphore_signal(barrier, inc=1, device_id=(nbr,),
                                device_id_type=_MESH)
        pl.semaphore_wait(barrier, 5)

        xb = x_ref[...].astype(jnp.bfloat16)
        wb = w_ref[...].astype(jnp.bfloat16)
        ccols = slice(CCOL, N)

        acc_ref[:, ccols] = jnp.dot(xb, wb[:, ccols],
                                    preferred_element_type=jnp.float32)
        cz_sends = []
        for k, (zd, pzd, slot) in enumerate(z_dests):
            czs[k] = (acc_ref[pl.ds(zd * PG, PG), ccols]
                      .astype(jnp.bfloat16))
            r = pltpu.make_async_remote_copy(
                src_ref=czs.at[k], dst_ref=czr.at[slot],
                send_sem=cz_ssem.at[k], recv_sem=cz_rsem.at[slot],
                device_id=(pzd,), device_id_type=_MESH)
            r.start()
            cz_sends.append(r)

        def strip_cfg(s):
            if s % 2 == 0:
                o1, o2 = jnp.minimum(cx, cd), jnp.maximum(cx, cd)
                return (px, py), (o1, o2), cy
            o1, o2 = jnp.minimum(cy, cd), jnp.maximum(cy, cd)
            return (py, px), (o1, o2), cx

        p1_rdmas, p2_rdmas, z_sends = [], [], []

        for s in range(NS):
            cols = slice(s * SW, (s + 1) * SW)
            (peer1, _), (o1, o2), _ = strip_cfg(s)
            acc_ref[:, cols] = jnp.dot(xb, wb[:, cols],
                                       preferred_element_type=jnp.float32)
            for zi in range(NZ):
                base = zi * NZ * BLK
                p1s[s, pl.ds(2 * zi * BLK, BLK), :] = (
                    acc_ref[pl.ds(base + o1 * BLK, BLK), cols]
                    .astype(jnp.bfloat16))
                p1s[s, pl.ds((2 * zi + 1) * BLK, BLK), :] = (
                    acc_ref[pl.ds(base + o2 * BLK, BLK), cols]
                    .astype(jnp.bfloat16))
            r = pltpu.make_async_remote_copy(
                src_ref=p1s.at[s], dst_ref=p1r.at[s],
                send_sem=p1_ssem.at[s], recv_sem=p1_rsem.at[s],
                device_id=(peer1,), device_id_type=_MESH)
            r.start()
            p1_rdmas.append(r)

        for s in range(NS):
            cols = slice(s * SW, (s + 1) * SW)
            (_, peer2), _, fwd = strip_cfg(s)
            j_c = (c > fwd).astype(jnp.int32)
            j_f = 1 - j_c
            p1_rdmas[s].wait()
            for zi in range(NZ):
                base = zi * NZ * BLK
                rc = pl.ds(base + c * BLK, BLK)
                acc_ref[rc, cols] = (
                    acc_ref[rc, cols]
                    + p1r[s, pl.ds((2 * zi + j_c) * BLK, BLK), :]
                    .astype(jnp.float32))
                p2s[s, pl.ds(zi * BLK, BLK), :] = (
                    acc_ref[pl.ds(base + fwd * BLK, BLK), cols]
                    + p1r[s, pl.ds((2 * zi + j_f) * BLK, BLK), :]
                    .astype(jnp.float32)).astype(jnp.bfloat16)
            r = pltpu.make_async_remote_copy(
                src_ref=p2s.at[s], dst_ref=p2r.at[s],
                send_sem=p2_ssem.at[s], recv_sem=p2_rsem.at[s],
                device_id=(peer2,), device_id_type=_MESH)
            r.start()
            p2_rdmas.append(r)

        for j in range(NZ - 1):
            pltpu.make_async_remote_copy(
                src_ref=czs.at[j], dst_ref=czr.at[j],
                send_sem=cz_ssem.at[j], recv_sem=cz_rsem.at[j],
                device_id=(p,), device_id_type=_MESH).wait_recv()
        mypg = pl.ds(z * PG, PG)
        acc_ref[mypg, ccols] = (
            acc_ref[mypg, ccols]
            + czr[0].astype(jnp.float32)
            + czr[1].astype(jnp.float32)
            + czr[2].astype(jnp.float32))
        co1, co2 = jnp.minimum(cy, cd), jnp.maximum(cy, cd)
        cp1s[pl.ds(0, BLK), :] = (
            acc_ref[pl.ds(z * PG + co1 * BLK, BLK), ccols]
            .astype(jnp.bfloat16))
        cp1s[pl.ds(BLK, BLK), :] = (
            acc_ref[pl.ds(z * PG + co2 * BLK, BLK), ccols]
            .astype(jnp.bfloat16))
        rdma_cp1 = pltpu.make_async_remote_copy(
            src_ref=cp1s, dst_ref=cp1r,
            send_sem=cp1_ssem.at[0], recv_sem=cp1_rsem.at[0],
            device_id=(py,), device_id_type=_MESH)
        rdma_cp1.start()

        for s in range(NS):
            cols = slice(s * SW, (s + 1) * SW)
            p2_rdmas[s].wait()
            for zi in range(NZ):
                rc = pl.ds((zi * NZ + c) * BLK, BLK)
                zfull[zi, :, pl.ds(s * SW, SW)] = (
                    acc_ref[rc, cols]
                    + p2r[s, pl.ds(zi * BLK, BLK), :].astype(jnp.float32)
                ).astype(jnp.bfloat16)
            for k, (zd, pzd, slot) in enumerate(z_dests):
                r = pltpu.make_async_remote_copy(
                    src_ref=zfull.at[zd, :, pl.ds(s * SW, SW)],
                    dst_ref=zdr.at[slot, :, pl.ds(s * SW, SW)],
                    send_sem=z_ssem.at[k * NS + s],
                    recv_sem=z_rsem.at[slot * NS + s],
                    device_id=(pzd,), device_id_type=_MESH)
                r.start()
                z_sends.append(r)

        cj_c = (c > cx).astype(jnp.int32)
        cj_f = 1 - cj_c
        rdma_cp1.wait()
        rcm = pl.ds(z * PG + c * BLK, BLK)
        acc_ref[rcm, ccols] = (
            acc_ref[rcm, ccols]
            + cp1r[pl.ds(cj_c * BLK, BLK), :].astype(jnp.float32))
        cp2s[...] = (
            acc_ref[pl.ds(z * PG + cx * BLK, BLK), ccols]
            + cp1r[pl.ds(cj_f * BLK, BLK), :].astype(jnp.float32)
        ).astype(jnp.bfloat16)
        rdma_cp2 = pltpu.make_async_remote_copy(
            src_ref=cp2s, dst_ref=cp2r,
            send_sem=cp2_ssem.at[0], recv_sem=cp2_rsem.at[0],
            device_id=(px,), device_id_type=_MESH)
        rdma_cp2.start()

        for s in range(NS):
            cols = slice(s * SW, (s + 1) * SW)
            fin = zfull[z, :, cols].astype(jnp.float32)
            for j in range(NZ - 1):
                pltpu.make_async_remote_copy(
                    src_ref=zfull.at[j, :, pl.ds(s * SW, SW)],
                    dst_ref=zdr.at[j, :, pl.ds(s * SW, SW)],
                    send_sem=z_ssem.at[j * NS + s],
                    recv_sem=z_rsem.at[j * NS + s],
                    device_id=(p,), device_id_type=_MESH).wait_recv()
                fin = fin + zdr[j, :, cols].astype(jnp.float32)
            out_ref[:, cols] = _gelu(fin)

        rdma_cp2.wait()
        cfin = acc_ref[rcm, ccols] + cp2r[...].astype(jnp.float32)
        out_ref[:, ccols] = _gelu(cfin)

        for r in cz_sends + z_sends:
            r.wait_send()

    return pl.pallas_call(
        body,
        out_shape=jax.ShapeDtypeStruct((BLK, n), jnp.float32),
        in_specs=[
            pl.BlockSpec(memory_space=pltpu.VMEM),
            pl.BlockSpec(memory_space=pltpu.VMEM),
        ],
        out_specs=pl.BlockSpec(memory_space=pltpu.VMEM),
        scratch_shapes=[
            pltpu.VMEM((m, n), jnp.float32),
            pltpu.VMEM((NS, 8 * BLK, SW), jnp.bfloat16),
            pltpu.VMEM((NS, 8 * BLK, SW), jnp.bfloat16),
            pltpu.VMEM((NS, 4 * BLK, SW), jnp.bfloat16),
            pltpu.VMEM((NS, 4 * BLK, SW), jnp.bfloat16),
            pltpu.VMEM((NZ, BLK, CCOL), jnp.bfloat16),
            pltpu.VMEM((NZ - 1, BLK, CCOL), jnp.bfloat16),
            pltpu.VMEM((NZ - 1, PG, N - CCOL), jnp.bfloat16),
            pltpu.VMEM((NZ - 1, PG, N - CCOL), jnp.bfloat16),
            pltpu.VMEM((2 * BLK, N - CCOL), jnp.bfloat16),
            pltpu.VMEM((2 * BLK, N - CCOL), jnp.bfloat16),
            pltpu.VMEM((BLK, N - CCOL), jnp.bfloat16),
            pltpu.VMEM((BLK, N - CCOL), jnp.bfloat16),
            pltpu.SemaphoreType.DMA((NS,)),
            pltpu.SemaphoreType.DMA((NS,)),
            pltpu.SemaphoreType.DMA((NS,)),
            pltpu.SemaphoreType.DMA((NS,)),
            pltpu.SemaphoreType.DMA(((NZ - 1) * NS,)),
            pltpu.SemaphoreType.DMA(((NZ - 1) * NS,)),
            pltpu.SemaphoreType.DMA((NZ - 1,)),
            pltpu.SemaphoreType.DMA((NZ - 1,)),
            pltpu.SemaphoreType.DMA((1,)),
            pltpu.SemaphoreType.DMA((1,)),
            pltpu.SemaphoreType.DMA((1,)),
            pltpu.SemaphoreType.DMA((1,)),
        ],
        compiler_params=pltpu.CompilerParams(collective_id=0),
    )(x, w_mat)
